# Initial kernel scaffold; baseline (speedup 1.0000x reference)
import jax
import jax.numpy as jnp
from jax.experimental import pallas as pl


def kernel(edge_index, u_S, W1, b1, gamma1, beta1, W2, b2, gamma2, beta2):
    raise NotImplementedError("write your pallas kernel here")


# speedup vs baseline: 13.9356x; 13.9356x over previous
import functools

import jax
import jax.numpy as jnp
from jax import lax
from jax.experimental import pallas as pl
from jax.experimental.pallas import tpu as pltpu
from jax.experimental.pallas import tpu_sc as plsc

N = 10000
E = 320000
D_IN = 128
D_HID = 128
D_OUT = 2
EPS = 1e-5

NC = 2
NS = 16
NW = NC * NS
EPW = E // NW
CH = 80
NCHUNK = EPW // CH
NPAD = 10240
RPT = NPAD // NS
DEGW = 16

_MESH = plsc.VectorSubcoreMesh(core_axis_name="c", subcore_axis_name="s")
_SC_PARAMS = pltpu.CompilerParams(use_tc_tiling_on_sc=False)


def _make_scatter_kernel(width):
    @functools.partial(
        pl.kernel,
        out_type=jax.ShapeDtypeStruct((NC * NPAD, width), jnp.float32),
        mesh=_MESH,
        compiler_params=_SC_PARAMS,
        scratch_types=[
            pltpu.VMEM((CH,), jnp.int32),
            pltpu.VMEM((CH,), jnp.int32),
            pltpu.VMEM((CH, width), jnp.float32),
            pltpu.VMEM_SHARED((NPAD, width), jnp.float32),
            pltpu.SemaphoreType.DMA,
        ],
    )
    def scatter_kernel(g_hbm, src_hbm, dst_hbm, zeros_hbm, out_hbm,
                       srcv, dstv, rows, acc, sem):
        c = lax.axis_index("c")
        s = lax.axis_index("s")
        wid = s * NC + c
        r0 = s * RPT
        pltpu.sync_copy(zeros_hbm.at[pl.ds(r0, RPT)], acc.at[pl.ds(r0, RPT)])
        plsc.subcore_barrier()
        base = wid * EPW

        def chunk(i, carry):
            b = base + i * CH
            pltpu.sync_copy(src_hbm.at[pl.ds(b, CH)], srcv)
            pltpu.sync_copy(dst_hbm.at[pl.ds(b, CH)], dstv)
            pltpu.async_copy(g_hbm.at[srcv], rows, sem).wait()
            pltpu.sync_copy(rows, acc.at[dstv], add=True)
            return carry

        lax.fori_loop(0, NCHUNK, chunk, 0)
        plsc.subcore_barrier()
        pltpu.sync_copy(acc.at[pl.ds(r0, RPT)],
                        out_hbm.at[pl.ds(c * NPAD + r0, RPT)])

    return scatter_kernel


_scatter128 = _make_scatter_kernel(D_HID)
_scatter16 = _make_scatter_kernel(DEGW)


@functools.partial(
    pl.kernel,
    out_type=jax.ShapeDtypeStruct((NC * NPAD, DEGW), jnp.float32),
    mesh=_MESH,
    compiler_params=_SC_PARAMS,
    scratch_types=[
        pltpu.VMEM((CH,), jnp.int32),
        pltpu.VMEM((CH, DEGW), jnp.float32),
        pltpu.VMEM_SHARED((NPAD, DEGW), jnp.float32),
    ],
)
def _deg_kernel(dst_hbm, ones_hbm, zeros_hbm, out_hbm, dstv, onesv, acc):
    c = lax.axis_index("c")
    s = lax.axis_index("s")
    wid = s * NC + c
    r0 = s * RPT
    pltpu.sync_copy(ones_hbm, onesv)
    pltpu.sync_copy(zeros_hbm.at[pl.ds(r0, RPT)], acc.at[pl.ds(r0, RPT)])
    plsc.subcore_barrier()
    base = wid * EPW

    def chunk(i, carry):
        pltpu.sync_copy(dst_hbm.at[pl.ds(base + i * CH, CH)], dstv)
        pltpu.sync_copy(onesv, acc.at[dstv], add=True)
        return carry

    lax.fori_loop(0, NCHUNK, chunk, 0)
    plsc.subcore_barrier()
    pltpu.sync_copy(acc.at[pl.ds(r0, RPT)],
                    out_hbm.at[pl.ds(c * NPAD + r0, RPT)])


def _batch_norm(x, gamma, beta):
    mean = jnp.mean(x, axis=0)
    var = jnp.mean((x - mean) ** 2, axis=0)
    return gamma * (x - mean) * lax.rsqrt(var + EPS) + beta


def kernel(edge_index, u_S, W1, b1, gamma1, beta1, W2, b2, gamma2, beta2):
    src = edge_index[0]
    dst = edge_index[1]
    zeros128 = jnp.zeros((NPAD, D_HID), jnp.float32)
    zeros16 = jnp.zeros((NPAD, DEGW), jnp.float32)
    ones_chunk = jnp.ones((CH, DEGW), jnp.float32)

    degp = _deg_kernel(dst, ones_chunk, zeros16)
    deg = degp[:N, 0] + degp[NPAD:NPAD + N, 0] + 1.0
    dinv = lax.rsqrt(deg)

    h1 = u_S @ W1
    g1 = dinv[:, None] * h1
    p1 = _scatter128(g1, src, dst, zeros128)
    x1 = dinv[:, None] * (p1[:N] + p1[NPAD:NPAD + N] + g1) + b1
    s1 = jax.nn.relu(_batch_norm(x1, gamma1, beta1))

    h2 = s1 @ W2
    g2 = dinv[:, None] * h2
    g2p = jnp.concatenate(
        [g2, jnp.zeros((N, DEGW - D_OUT), jnp.float32)], axis=1)
    p2 = _scatter16(g2p, src, dst, zeros16)
    x2 = dinv[:, None] * (p2[:N, :D_OUT] + p2[NPAD:NPAD + N, :D_OUT] + g2) + b2
    s2 = _batch_norm(x2, gamma2, beta2)
    return (jax.nn.sigmoid(s2), jax.nn.softmax(s2, axis=1))


# speedup vs baseline: 32.1505x; 2.3071x over previous
import functools

import jax
import jax.numpy as jnp
from jax import lax
from jax.experimental import pallas as pl
from jax.experimental.pallas import tpu as pltpu
from jax.experimental.pallas import tpu_sc as plsc

N = 10000
E = 320000
D_IN = 128
D_HID = 128
D_OUT = 2
EPS = 1e-5

NC = 2
NS = 16
NW = NC * NS
EPW = E // NW
CH = 100
NCHUNK = EPW // CH
NPAD = 10240
RPT = NPAD // NS
DEGW = 16

_MESH = plsc.VectorSubcoreMesh(core_axis_name="c", subcore_axis_name="s")
_SC_PARAMS = pltpu.CompilerParams(use_tc_tiling_on_sc=False)


def _make_scatter_kernel(width):
    @functools.partial(
        pl.kernel,
        out_type=jax.ShapeDtypeStruct((NC * NPAD, width), jnp.float32),
        mesh=_MESH,
        compiler_params=_SC_PARAMS,
        scratch_types=[
            pltpu.VMEM((NCHUNK, CH), jnp.int32),
            pltpu.VMEM((NCHUNK, CH), jnp.int32),
            pltpu.VMEM((CH, width), jnp.float32),
            pltpu.VMEM((CH, width), jnp.float32),
            pltpu.VMEM_SHARED((NPAD, width), jnp.float32),
            pltpu.SemaphoreType.DMA,
            pltpu.SemaphoreType.DMA,
        ],
    )
    def scatter_kernel(g_hbm, src_hbm, dst_hbm, zeros_hbm, out_hbm,
                       srcv, dstv, rows0, rows1, acc, sem0, sem1):
        c = lax.axis_index("c")
        s = lax.axis_index("s")
        wid = s * NC + c
        r0 = s * RPT
        pltpu.sync_copy(zeros_hbm.at[pl.ds(r0, RPT)], acc.at[pl.ds(r0, RPT)])
        pltpu.sync_copy(src_hbm.at[wid], srcv)
        pltpu.sync_copy(dst_hbm.at[wid], dstv)
        plsc.subcore_barrier()

        pltpu.async_copy(g_hbm.at[srcv.at[0]], rows0, sem0)

        def pair(t, carry):
            j0 = 2 * t
            j1 = j0 + 1
            pltpu.async_copy(g_hbm.at[srcv.at[j1]], rows1, sem1)
            pltpu.make_async_copy(g_hbm.at[srcv.at[j0]], rows0, sem0).wait()
            pltpu.sync_copy(rows0, acc.at[dstv.at[j0]], add=True)

            @pl.when(j1 + 1 < NCHUNK)
            def _():
                pltpu.async_copy(g_hbm.at[srcv.at[j1 + 1]], rows0, sem0)

            pltpu.make_async_copy(g_hbm.at[srcv.at[j1]], rows1, sem1).wait()
            pltpu.sync_copy(rows1, acc.at[dstv.at[j1]], add=True)
            return carry

        lax.fori_loop(0, NCHUNK // 2, pair, 0)
        plsc.subcore_barrier()
        pltpu.sync_copy(acc.at[pl.ds(r0, RPT)],
                        out_hbm.at[pl.ds(c * NPAD + r0, RPT)])

    return scatter_kernel


_scatter128 = _make_scatter_kernel(D_HID)
_scatter16 = _make_scatter_kernel(DEGW)


@functools.partial(
    pl.kernel,
    out_type=jax.ShapeDtypeStruct((NC * NPAD, DEGW), jnp.float32),
    mesh=_MESH,
    compiler_params=_SC_PARAMS,
    scratch_types=[
        pltpu.VMEM((NCHUNK, CH), jnp.int32),
        pltpu.VMEM((CH, DEGW), jnp.float32),
        pltpu.VMEM_SHARED((NPAD, DEGW), jnp.float32),
    ],
)
def _deg_kernel(dst_hbm, ones_hbm, zeros_hbm, out_hbm, dstv, onesv, acc):
    c = lax.axis_index("c")
    s = lax.axis_index("s")
    wid = s * NC + c
    r0 = s * RPT
    pltpu.sync_copy(ones_hbm, onesv)
    pltpu.sync_copy(zeros_hbm.at[pl.ds(r0, RPT)], acc.at[pl.ds(r0, RPT)])
    pltpu.sync_copy(dst_hbm.at[wid], dstv)
    plsc.subcore_barrier()

    def chunk(i, carry):
        pltpu.sync_copy(onesv, acc.at[dstv.at[i]], add=True)
        return carry

    lax.fori_loop(0, NCHUNK, chunk, 0)
    plsc.subcore_barrier()
    pltpu.sync_copy(acc.at[pl.ds(r0, RPT)],
                    out_hbm.at[pl.ds(c * NPAD + r0, RPT)])


def _batch_norm(x, gamma, beta):
    mean = jnp.mean(x, axis=0)
    var = jnp.mean((x - mean) ** 2, axis=0)
    return gamma * (x - mean) * lax.rsqrt(var + EPS) + beta


def kernel(edge_index, u_S, W1, b1, gamma1, beta1, W2, b2, gamma2, beta2):
    src = edge_index[0].reshape(NW, NCHUNK, CH)
    dst = edge_index[1].reshape(NW, NCHUNK, CH)
    zeros128 = jnp.zeros((NPAD, D_HID), jnp.float32)
    zeros16 = jnp.zeros((NPAD, DEGW), jnp.float32)
    ones_chunk = jnp.ones((CH, DEGW), jnp.float32)

    degp = _deg_kernel(dst, ones_chunk, zeros16)
    deg = degp[:N, 0] + degp[NPAD:NPAD + N, 0] + 1.0
    dinv = lax.rsqrt(deg)

    h1 = u_S @ W1
    g1 = dinv[:, None] * h1
    p1 = _scatter128(g1, src, dst, zeros128)
    x1 = dinv[:, None] * (p1[:N] + p1[NPAD:NPAD + N] + g1) + b1
    s1 = jax.nn.relu(_batch_norm(x1, gamma1, beta1))

    h2 = s1 @ W2
    g2 = dinv[:, None] * h2
    g2p = jnp.concatenate(
        [g2, jnp.zeros((N, DEGW - D_OUT), jnp.float32)], axis=1)
    p2 = _scatter16(g2p, src, dst, zeros16)
    x2 = dinv[:, None] * (p2[:N, :D_OUT] + p2[NPAD:NPAD + N, :D_OUT] + g2) + b2
    s2 = _batch_norm(x2, gamma2, beta2)
    return (jax.nn.sigmoid(s2), jax.nn.softmax(s2, axis=1))
